# initial kernel scaffold (unmeasured)
import jax
import jax.numpy as jnp
from jax import lax
from jax.experimental import pallas as pl
from jax.experimental.pallas import tpu as pltpu

N_DEV = 4
M_BLK = 2048
D = 2048

_ANY = getattr(pltpu, "ANY", None) or pltpu.MemorySpace.ANY
_CompilerParams = getattr(pltpu, "CompilerParams", None) or pltpu.TPUCompilerParams


def kernel(partial, gamma):
    x = partial.reshape(N_DEV * M_BLK, D)
    g = gamma.reshape(1, D)

    def body(x_ref, g_ref, out_ref, comm_out, comm_in, blk,
             load_sem, send_sems, recv_sems):
        my_x = lax.axis_index("x")
        my_y = lax.axis_index("y")
        my_z = lax.axis_index("z")
        left = lax.rem(my_y + (N_DEV - 1), N_DEV)
        right = lax.rem(my_y + 1, N_DEV)

        barrier_sem = pltpu.get_barrier_semaphore()
        for nbr in (left, right):
            pl.semaphore_signal(
                barrier_sem, inc=1,
                device_id=(my_x, nbr, my_z),
                device_id_type=pl.DeviceIdType.MESH,
            )
        pl.semaphore_wait(barrier_sem, 2)

        def load_block(c):
            cp = pltpu.make_async_copy(
                x_ref.at[pl.ds(c * M_BLK, M_BLK), :], blk, load_sem)
            cp.start()
            cp.wait()

        for h in range(N_DEV - 1):
            c = lax.rem(my_y + (N_DEV - 1 - h), N_DEV)
            load_block(c)
            if h == 0:
                comm_out[...] = blk[...].astype(jnp.bfloat16)
            else:
                comm_out[...] = (
                    comm_in[h - 1].astype(jnp.float32) + blk[...]
                ).astype(jnp.bfloat16)
            rdma = pltpu.make_async_remote_copy(
                src_ref=comm_out,
                dst_ref=comm_in.at[h],
                send_sem=send_sems.at[h],
                recv_sem=recv_sems.at[h],
                device_id=(my_x, right, my_z),
                device_id_type=pl.DeviceIdType.MESH,
            )
            rdma.start()
            rdma.wait()

        load_block(my_y)
        y = comm_in[N_DEV - 2].astype(jnp.float32) + blk[...]
        ms = jnp.mean(y * y, axis=-1, keepdims=True)
        out_ref[...] = y * lax.rsqrt(ms + 1e-6) * g_ref[...]

    return pl.pallas_call(
        body,
        out_shape=jax.ShapeDtypeStruct((M_BLK, D), jnp.float32),
        in_specs=[
            pl.BlockSpec(memory_space=_ANY),
            pl.BlockSpec(memory_space=pltpu.VMEM),
        ],
        out_specs=pl.BlockSpec(memory_space=pltpu.VMEM),
        scratch_shapes=[
            pltpu.VMEM((M_BLK, D), jnp.bfloat16),
            pltpu.VMEM((N_DEV - 1, M_BLK, D), jnp.bfloat16),
            pltpu.VMEM((M_BLK, D), jnp.float32),
            pltpu.SemaphoreType.DMA,
            pltpu.SemaphoreType.DMA((N_DEV - 1,)),
            pltpu.SemaphoreType.DMA((N_DEV - 1,)),
        ],
        compiler_params=_CompilerParams(collective_id=0),
    )(x, g)


# baseline (device time: 343682 ns/iter reference)
import jax
import jax.numpy as jnp
from jax import lax
from jax.experimental import pallas as pl
from jax.experimental.pallas import tpu as pltpu

N_DEV = 4
M_BLK = 2048
D = 2048
T = 512
N_T = M_BLK // T


def kernel(partial, gamma):
    x = partial.reshape(N_DEV * M_BLK, D)
    g = gamma.reshape(1, D)

    def body(x_ref, g_ref, out_ref, send_buf, comm_in, tile, out_stage,
             load_sem, store_sem, send_sems, recv_sems):
        my_x = lax.axis_index("x")
        my_y = lax.axis_index("y")
        my_z = lax.axis_index("z")
        left = lax.rem(my_y + (N_DEV - 1), N_DEV)
        right = lax.rem(my_y + 1, N_DEV)

        barrier_sem = pltpu.get_barrier_semaphore()
        for nbr in (left, right):
            pl.semaphore_signal(
                barrier_sem, inc=1,
                device_id=(my_x, nbr, my_z),
                device_id_type=pl.DeviceIdType.MESH,
            )
        pl.semaphore_wait(barrier_sem, 2)

        def load_tile(c, t):
            cp = pltpu.make_async_copy(
                x_ref.at[pl.ds(c * M_BLK + t * T, T), :], tile, load_sem)
            cp.start()
            cp.wait()

        for h in range(N_DEV - 1):
            c = lax.rem(my_y + (N_DEV - 1 - h), N_DEV)
            for t in range(N_T):
                load_tile(c, t)
                sl = pl.ds(t * T, T)
                if h == 0:
                    send_buf[sl, :] = tile[...].astype(jnp.bfloat16)
                else:
                    send_buf[sl, :] = (
                        comm_in[h - 1, sl, :].astype(jnp.float32) + tile[...]
                    ).astype(jnp.bfloat16)
            rdma = pltpu.make_async_remote_copy(
                src_ref=send_buf,
                dst_ref=comm_in.at[h],
                send_sem=send_sems.at[h],
                recv_sem=recv_sems.at[h],
                device_id=(my_x, right, my_z),
                device_id_type=pl.DeviceIdType.MESH,
            )
            rdma.start()
            rdma.wait()

        for t in range(N_T):
            load_tile(my_y, t)
            sl = pl.ds(t * T, T)
            y = comm_in[N_DEV - 2, sl, :].astype(jnp.float32) + tile[...]
            ms = jnp.mean(y * y, axis=-1, keepdims=True)
            out_stage[...] = y * lax.rsqrt(ms + 1e-6) * g_ref[...]
            cp = pltpu.make_async_copy(out_stage, out_ref.at[sl, :], store_sem)
            cp.start()
            cp.wait()

    return pl.pallas_call(
        body,
        out_shape=jax.ShapeDtypeStruct((M_BLK, D), jnp.float32),
        in_specs=[
            pl.BlockSpec(memory_space=pl.ANY),
            pl.BlockSpec(memory_space=pltpu.MemorySpace.VMEM),
        ],
        out_specs=pl.BlockSpec(memory_space=pl.ANY),
        scratch_shapes=[
            pltpu.VMEM((M_BLK, D), jnp.bfloat16),
            pltpu.VMEM((N_DEV - 1, M_BLK, D), jnp.bfloat16),
            pltpu.VMEM((T, D), jnp.float32),
            pltpu.VMEM((T, D), jnp.float32),
            pltpu.SemaphoreType.DMA,
            pltpu.SemaphoreType.DMA,
            pltpu.SemaphoreType.DMA((N_DEV - 1,)),
            pltpu.SemaphoreType.DMA((N_DEV - 1,)),
        ],
        compiler_params=pltpu.CompilerParams(
            collective_id=0, vmem_limit_bytes=60 * 1024 * 1024),
    )(x, g)


# device time: 164888 ns/iter; 2.0843x vs baseline; 2.0843x over previous
import jax
import jax.numpy as jnp
from jax import lax
from jax.experimental import pallas as pl
from jax.experimental.pallas import tpu as pltpu

N_Y = 4
N_P = 8
M_BLK = 2048
D = 2048
W = D // N_P
T = 512


def kernel(partial, gamma):
    x = partial.reshape(N_Y * M_BLK, D)
    g = gamma.reshape(1, D)

    def body(x_ref, g_ref, out_ref, rs_send, rs_in, tile, y_all, out_stage,
             load_sem, store_sem, rs_send_sems, rs_recv_sems,
             ag_send_sems, ag_recv_sems):
        my_x = lax.axis_index("x")
        my_y = lax.axis_index("y")
        my_z = lax.axis_index("z")
        y_left = lax.rem(my_y + (N_Y - 1), N_Y)
        y_right = lax.rem(my_y + 1, N_Y)

        p = jnp.where(my_x == 0, my_z, N_P - 1 - my_z)

        def pos_xz(o):
            return jnp.where(o < 4, 0, 1), jnp.where(o < 4, o, N_P - 1 - o)

        def col_of_pos(o):
            ox, oz = pos_xz(o)
            return (ox * 4 + oz) * W

        p_next = lax.rem(p + 1, N_P)
        p_prev = lax.rem(p + (N_P - 1), N_P)
        nx, nz = pos_xz(p_next)
        px, pz = pos_xz(p_prev)
        col0 = (my_x * 4 + my_z) * W

        barrier_sem = pltpu.get_barrier_semaphore()
        for dev in ((my_x, y_left, my_z), (my_x, y_right, my_z),
                    (nx, my_y, nz), (px, my_y, pz)):
            pl.semaphore_signal(
                barrier_sem, inc=1,
                device_id=dev, device_id_type=pl.DeviceIdType.MESH,
            )
        pl.semaphore_wait(barrier_sem, 4)

        def load_chunk(c):
            cp = pltpu.make_async_copy(
                x_ref.at[pl.ds(c * M_BLK, M_BLK), pl.ds(col0, W)],
                tile, load_sem)
            cp.start()
            cp.wait()

        for h in range(N_Y - 1):
            c = lax.rem(my_y + (N_Y - 1 - h), N_Y)
            load_chunk(c)
            if h == 0:
                rs_send[...] = tile[...].astype(jnp.bfloat16)
            else:
                rs_send[...] = (
                    rs_in[h - 1].astype(jnp.float32) + tile[...]
                ).astype(jnp.bfloat16)
            rdma = pltpu.make_async_remote_copy(
                src_ref=rs_send,
                dst_ref=rs_in.at[h],
                send_sem=rs_send_sems.at[h],
                recv_sem=rs_recv_sems.at[h],
                device_id=(my_x, y_right, my_z),
                device_id_type=pl.DeviceIdType.MESH,
            )
            rdma.start()
            rdma.wait()

        load_chunk(my_y)
        y_all[:, pl.ds(col0, W)] = (
            rs_in[N_Y - 2].astype(jnp.float32) + tile[...]
        ).astype(jnp.bfloat16)

        for k in range(N_P - 1):
            o = lax.rem(p + (N_P - k), N_P)
            oc = col_of_pos(o)
            rdma = pltpu.make_async_remote_copy(
                src_ref=y_all.at[:, pl.ds(oc, W)],
                dst_ref=y_all.at[:, pl.ds(oc, W)],
                send_sem=ag_send_sems.at[k],
                recv_sem=ag_recv_sems.at[k],
                device_id=(nx, my_y, nz),
                device_id_type=pl.DeviceIdType.MESH,
            )
            rdma.start()
            rdma.wait()

        for t in range(M_BLK // T):
            sl = pl.ds(t * T, T)
            y = y_all[sl, :].astype(jnp.float32)
            ms = jnp.mean(y * y, axis=-1, keepdims=True)
            out_stage[...] = y * lax.rsqrt(ms + 1e-6) * g_ref[...]
            cp = pltpu.make_async_copy(out_stage, out_ref.at[sl, :], store_sem)
            cp.start()
            cp.wait()

    return pl.pallas_call(
        body,
        out_shape=jax.ShapeDtypeStruct((M_BLK, D), jnp.float32),
        in_specs=[
            pl.BlockSpec(memory_space=pl.ANY),
            pl.BlockSpec(memory_space=pltpu.MemorySpace.VMEM),
        ],
        out_specs=pl.BlockSpec(memory_space=pl.ANY),
        scratch_shapes=[
            pltpu.VMEM((M_BLK, W), jnp.bfloat16),
            pltpu.VMEM((N_Y - 1, M_BLK, W), jnp.bfloat16),
            pltpu.VMEM((M_BLK, W), jnp.float32),
            pltpu.VMEM((M_BLK, D), jnp.bfloat16),
            pltpu.VMEM((T, D), jnp.float32),
            pltpu.SemaphoreType.DMA,
            pltpu.SemaphoreType.DMA,
            pltpu.SemaphoreType.DMA((N_Y - 1,)),
            pltpu.SemaphoreType.DMA((N_Y - 1,)),
            pltpu.SemaphoreType.DMA((N_P - 1,)),
            pltpu.SemaphoreType.DMA((N_P - 1,)),
        ],
        compiler_params=pltpu.CompilerParams(
            collective_id=0, vmem_limit_bytes=60 * 1024 * 1024),
    )(x, g)


# device time: 127318 ns/iter; 2.6994x vs baseline; 1.2951x over previous
import jax
import jax.numpy as jnp
from jax import lax
from jax.experimental import pallas as pl
from jax.experimental.pallas import tpu as pltpu

N_Y = 4
N_P = 8
M_BLK = 2048
D = 2048
W = D // N_P
T = 512


def kernel(partial, gamma):
    x = partial.reshape(N_Y * M_BLK, D)
    g = gamma.reshape(1, D)

    def body(x_ref, g_ref, out_ref, rs_send, rs_in, tile, y_all, out_stage,
             load_sem, store_sem, rs_send_sems, rs_recv_sems,
             ag_send_sems, ag_recv_sems):
        my_x = lax.axis_index("x")
        my_y = lax.axis_index("y")
        my_z = lax.axis_index("z")
        y_left = lax.rem(my_y + (N_Y - 1), N_Y)
        y_right = lax.rem(my_y + 1, N_Y)

        p = jnp.where(my_x == 0, my_z, N_P - 1 - my_z)

        def pos_xz(o):
            return jnp.where(o < 4, 0, 1), jnp.where(o < 4, o, N_P - 1 - o)

        def col_of_pos(o):
            ox, oz = pos_xz(o)
            return (ox * 4 + oz) * W

        p_next = lax.rem(p + 1, N_P)
        p_prev = lax.rem(p + (N_P - 1), N_P)
        nx, nz = pos_xz(p_next)
        px, pz = pos_xz(p_prev)
        col0 = (my_x * 4 + my_z) * W

        barrier_sem = pltpu.get_barrier_semaphore()
        for dev in ((my_x, y_left, my_z), (my_x, y_right, my_z),
                    (nx, my_y, nz), (px, my_y, pz)):
            pl.semaphore_signal(
                barrier_sem, inc=1,
                device_id=dev, device_id_type=pl.DeviceIdType.MESH,
            )
        pl.semaphore_wait(barrier_sem, 4)

        def load_chunk(c):
            cp = pltpu.make_async_copy(
                x_ref.at[pl.ds(c * M_BLK, M_BLK), pl.ds(col0, W)],
                tile, load_sem)
            cp.start()
            cp.wait()

        for h in range(N_Y - 1):
            c = lax.rem(my_y + (N_Y - 1 - h), N_Y)
            load_chunk(c)
            if h == 0:
                rs_send[...] = tile[...].astype(jnp.bfloat16)
            else:
                rs_send[...] = (
                    rs_in[h - 1].astype(jnp.float32) + tile[...]
                ).astype(jnp.bfloat16)
            rdma = pltpu.make_async_remote_copy(
                src_ref=rs_send,
                dst_ref=rs_in.at[h],
                send_sem=rs_send_sems.at[h],
                recv_sem=rs_recv_sems.at[h],
                device_id=(my_x, y_right, my_z),
                device_id_type=pl.DeviceIdType.MESH,
            )
            rdma.start()
            rdma.wait()

        load_chunk(my_y)
        y_all[:, pl.ds(col0, W)] = (
            rs_in[N_Y - 2].astype(jnp.float32) + tile[...]
        ).astype(jnp.bfloat16)

        Wh = W // 2
        for k in range(N_P - 1):
            o_cw = lax.rem(p + (N_P - k), N_P)
            o_ccw = lax.rem(p + k, N_P)
            c_cw = col_of_pos(o_cw)
            c_ccw = col_of_pos(o_ccw) + Wh
            cw = pltpu.make_async_remote_copy(
                src_ref=y_all.at[:, pl.ds(c_cw, Wh)],
                dst_ref=y_all.at[:, pl.ds(c_cw, Wh)],
                send_sem=ag_send_sems.at[k, 0],
                recv_sem=ag_recv_sems.at[k, 0],
                device_id=(nx, my_y, nz),
                device_id_type=pl.DeviceIdType.MESH,
            )
            ccw = pltpu.make_async_remote_copy(
                src_ref=y_all.at[:, pl.ds(c_ccw, Wh)],
                dst_ref=y_all.at[:, pl.ds(c_ccw, Wh)],
                send_sem=ag_send_sems.at[k, 1],
                recv_sem=ag_recv_sems.at[k, 1],
                device_id=(px, my_y, pz),
                device_id_type=pl.DeviceIdType.MESH,
            )
            cw.start()
            ccw.start()
            cw.wait()
            ccw.wait()

        for t in range(M_BLK // T):
            sl = pl.ds(t * T, T)
            y = y_all[sl, :].astype(jnp.float32)
            ms = jnp.mean(y * y, axis=-1, keepdims=True)
            out_stage[...] = y * lax.rsqrt(ms + 1e-6) * g_ref[...]
            cp = pltpu.make_async_copy(out_stage, out_ref.at[sl, :], store_sem)
            cp.start()
            cp.wait()

    return pl.pallas_call(
        body,
        out_shape=jax.ShapeDtypeStruct((M_BLK, D), jnp.float32),
        in_specs=[
            pl.BlockSpec(memory_space=pl.ANY),
            pl.BlockSpec(memory_space=pltpu.MemorySpace.VMEM),
        ],
        out_specs=pl.BlockSpec(memory_space=pl.ANY),
        scratch_shapes=[
            pltpu.VMEM((M_BLK, W), jnp.bfloat16),
            pltpu.VMEM((N_Y - 1, M_BLK, W), jnp.bfloat16),
            pltpu.VMEM((M_BLK, W), jnp.float32),
            pltpu.VMEM((M_BLK, D), jnp.bfloat16),
            pltpu.VMEM((T, D), jnp.float32),
            pltpu.SemaphoreType.DMA,
            pltpu.SemaphoreType.DMA,
            pltpu.SemaphoreType.DMA((N_Y - 1,)),
            pltpu.SemaphoreType.DMA((N_Y - 1,)),
            pltpu.SemaphoreType.DMA((N_P - 1, 2)),
            pltpu.SemaphoreType.DMA((N_P - 1, 2)),
        ],
        compiler_params=pltpu.CompilerParams(
            collective_id=0, vmem_limit_bytes=60 * 1024 * 1024),
    )(x, g)


# device time: 118688 ns/iter; 2.8957x vs baseline; 1.0727x over previous
import jax
import jax.numpy as jnp
from jax import lax
from jax.experimental import pallas as pl
from jax.experimental.pallas import tpu as pltpu

N_Y = 4
N_P = 8
M_BLK = 2048
D = 2048
W = D // N_P
Wh = W // 2
T = 512


def kernel(partial, gamma):
    x = partial.reshape(N_Y * M_BLK, D)
    g = gamma.reshape(1, D)

    def body(x_ref, g_ref, out_ref, rs_send, rs_in, tile, y_all, ssq,
             out_stage, load_sems, store_sems, rs_send_sems, rs_recv_sems,
             ag_send_sems, ag_recv_sems):
        my_x = lax.axis_index("x")
        my_y = lax.axis_index("y")
        my_z = lax.axis_index("z")
        y_left = lax.rem(my_y + (N_Y - 1), N_Y)
        y_right = lax.rem(my_y + 1, N_Y)

        p = jnp.where(my_x == 0, my_z, N_P - 1 - my_z)

        def pos_xz(o):
            return jnp.where(o < 4, 0, 1), jnp.where(o < 4, o, N_P - 1 - o)

        def col_of_pos(o):
            ox, oz = pos_xz(o)
            return (ox * 4 + oz) * W

        p_next = lax.rem(p + 1, N_P)
        p_prev = lax.rem(p + (N_P - 1), N_P)
        nx, nz = pos_xz(p_next)
        px, pz = pos_xz(p_prev)
        col0 = (my_x * 4 + my_z) * W

        barrier_sem = pltpu.get_barrier_semaphore()
        for dev in ((my_x, y_left, my_z), (my_x, y_right, my_z),
                    (nx, my_y, nz), (px, my_y, pz)):
            pl.semaphore_signal(
                barrier_sem, inc=1,
                device_id=dev, device_id_type=pl.DeviceIdType.MESH,
            )
        pl.semaphore_wait(barrier_sem, 4)

        def start_load(c, slot):
            cp = pltpu.make_async_copy(
                x_ref.at[pl.ds(c * M_BLK, M_BLK), pl.ds(col0, W)],
                tile.at[slot], load_sems.at[slot])
            cp.start()
            return cp

        def wait_load(slot):
            pltpu.make_async_copy(
                x_ref.at[pl.ds(0, M_BLK), pl.ds(col0, W)],
                tile.at[slot], load_sems.at[slot]).wait()

        start_load(lax.rem(my_y + (N_Y - 1), N_Y), 0)
        prev_rdma = None
        for h in range(N_Y - 1):
            wait_load(h % 2)
            if h == 0:
                rs_send[...] = tile[0].astype(jnp.bfloat16)
            else:
                prev_rdma.wait_recv()
                prev_rdma.wait_send()
                rs_send[...] = (
                    rs_in[h - 1].astype(jnp.float32) + tile[h % 2]
                ).astype(jnp.bfloat16)
            rdma = pltpu.make_async_remote_copy(
                src_ref=rs_send,
                dst_ref=rs_in.at[h],
                send_sem=rs_send_sems.at[h],
                recv_sem=rs_recv_sems.at[h],
                device_id=(my_x, y_right, my_z),
                device_id_type=pl.DeviceIdType.MESH,
            )
            rdma.start()
            c_next = lax.rem(my_y + (N_Y - 2 - h), N_Y) if h < N_Y - 2 else my_y
            start_load(c_next, (h + 1) % 2)
            prev_rdma = rdma

        prev_rdma.wait_recv()
        prev_rdma.wait_send()
        wait_load((N_Y - 1) % 2)
        y_mine = rs_in[N_Y - 2].astype(jnp.float32) + tile[(N_Y - 1) % 2]
        y_all[:, pl.ds(col0, W)] = y_mine.astype(jnp.bfloat16)
        ssq[...] = jnp.sum(y_mine * y_mine, axis=-1, keepdims=True)

        def half_ssq(c):
            v = y_all[:, pl.ds(c, Wh)].astype(jnp.float32)
            return jnp.sum(v * v, axis=-1, keepdims=True)

        recv_cols = None
        for k in range(N_P - 1):
            o_cw = lax.rem(p + (N_P - k), N_P)
            o_ccw = lax.rem(p + k, N_P)
            cw = pltpu.make_async_remote_copy(
                src_ref=y_all.at[:, pl.ds(col_of_pos(o_cw), Wh)],
                dst_ref=y_all.at[:, pl.ds(col_of_pos(o_cw), Wh)],
                send_sem=ag_send_sems.at[k, 0],
                recv_sem=ag_recv_sems.at[k, 0],
                device_id=(nx, my_y, nz),
                device_id_type=pl.DeviceIdType.MESH,
            )
            ccw = pltpu.make_async_remote_copy(
                src_ref=y_all.at[:, pl.ds(col_of_pos(o_ccw) + Wh, Wh)],
                dst_ref=y_all.at[:, pl.ds(col_of_pos(o_ccw) + Wh, Wh)],
                send_sem=ag_send_sems.at[k, 1],
                recv_sem=ag_recv_sems.at[k, 1],
                device_id=(px, my_y, pz),
                device_id_type=pl.DeviceIdType.MESH,
            )
            cw.start()
            ccw.start()
            if recv_cols is not None:
                ssq[...] = ssq[...] + half_ssq(recv_cols[0]) + half_ssq(
                    recv_cols[1])
            cw.wait()
            ccw.wait()
            recv_cols = (
                col_of_pos(lax.rem(p + (N_P - 1 - k), N_P)),
                col_of_pos(lax.rem(p + k + 1, N_P)) + Wh,
            )
        ssq[...] = ssq[...] + half_ssq(recv_cols[0]) + half_ssq(recv_cols[1])

        rstd = lax.rsqrt(ssq[...] * (1.0 / D) + 1e-6)
        for t in range(M_BLK // T):
            sl = pl.ds(t * T, T)
            slot = t % 2
            if t >= 2:
                pltpu.make_async_copy(
                    out_stage.at[slot], out_ref.at[sl, :],
                    store_sems.at[slot]).wait()
            out_stage[slot] = (
                y_all[sl, :].astype(jnp.float32)
                * rstd[t * T:(t + 1) * T] * g_ref[...])
            cp = pltpu.make_async_copy(
                out_stage.at[slot], out_ref.at[sl, :], store_sems.at[slot])
            cp.start()
        for slot in range(2):
            pltpu.make_async_copy(
                out_stage.at[slot], out_ref.at[pl.ds(0, T), :],
                store_sems.at[slot]).wait()

    return pl.pallas_call(
        body,
        out_shape=jax.ShapeDtypeStruct((M_BLK, D), jnp.float32),
        in_specs=[
            pl.BlockSpec(memory_space=pl.ANY),
            pl.BlockSpec(memory_space=pltpu.MemorySpace.VMEM),
        ],
        out_specs=pl.BlockSpec(memory_space=pl.ANY),
        scratch_shapes=[
            pltpu.VMEM((M_BLK, W), jnp.bfloat16),
            pltpu.VMEM((N_Y - 1, M_BLK, W), jnp.bfloat16),
            pltpu.VMEM((2, M_BLK, W), jnp.float32),
            pltpu.VMEM((M_BLK, D), jnp.bfloat16),
            pltpu.VMEM((M_BLK, 1), jnp.float32),
            pltpu.VMEM((2, T, D), jnp.float32),
            pltpu.SemaphoreType.DMA((2,)),
            pltpu.SemaphoreType.DMA((2,)),
            pltpu.SemaphoreType.DMA((N_Y - 1,)),
            pltpu.SemaphoreType.DMA((N_Y - 1,)),
            pltpu.SemaphoreType.DMA((N_P - 1, 2)),
            pltpu.SemaphoreType.DMA((N_P - 1, 2)),
        ],
        compiler_params=pltpu.CompilerParams(
            collective_id=0, vmem_limit_bytes=60 * 1024 * 1024),
    )(x, g)


# device time: 114320 ns/iter; 3.0063x vs baseline; 1.0382x over previous
import jax
import jax.numpy as jnp
from jax import lax
from jax.experimental import pallas as pl
from jax.experimental.pallas import tpu as pltpu

N_Y = 4
N_P = 8
M_BLK = 2048
D = 2048
W = D // N_P
GW = W // 2
GH = GW // 2
T = 512


def kernel(partial, gamma):
    x = partial.reshape(N_Y * M_BLK, D)
    g = gamma.reshape(1, D)

    def body(x_ref, g_ref, out_ref, rs_send, rs_in, tile, y_all, ssq,
             out_stage, load_sems, store_sems, rs_send_sems, rs_recv_sems,
             ag_send_sems, ag_recv_sems):
        my_x = lax.axis_index("x")
        my_y = lax.axis_index("y")
        my_z = lax.axis_index("z")
        y_left = lax.rem(my_y + (N_Y - 1), N_Y)
        y_right = lax.rem(my_y + 1, N_Y)

        p = jnp.where(my_x == 0, my_z, N_P - 1 - my_z)

        def pos_xz(o):
            return jnp.where(o < 4, 0, 1), jnp.where(o < 4, o, N_P - 1 - o)

        def col_of_pos(o):
            ox, oz = pos_xz(o)
            return (ox * 4 + oz) * W

        p_next = lax.rem(p + 1, N_P)
        p_prev = lax.rem(p + (N_P - 1), N_P)
        nx, nz = pos_xz(p_next)
        px, pz = pos_xz(p_prev)
        col0 = (my_x * 4 + my_z) * W

        barrier_sem = pltpu.get_barrier_semaphore()
        for dev in ((my_x, y_left, my_z), (my_x, y_right, my_z),
                    (nx, my_y, nz), (px, my_y, pz)):
            pl.semaphore_signal(
                barrier_sem, inc=1,
                device_id=dev, device_id_type=pl.DeviceIdType.MESH,
            )
        pl.semaphore_wait(barrier_sem, 4)

        def rs_chunk(h):
            return lax.rem(my_y + (N_Y - 1 - h), N_Y)

        def load_start(grp, slot, c):
            pltpu.make_async_copy(
                x_ref.at[pl.ds(c * M_BLK, M_BLK),
                         pl.ds(col0 + grp * GW, GW)],
                tile.at[grp, slot], load_sems.at[grp, slot]).start()

        def load_wait(grp, slot):
            pltpu.make_async_copy(
                x_ref.at[pl.ds(0, M_BLK), pl.ds(col0, GW)],
                tile.at[grp, slot], load_sems.at[grp, slot]).wait()

        def rs_hop(grp, h, prev):
            load_wait(grp, h % 2)
            if h == 0:
                rs_send[grp] = tile[grp, 0].astype(jnp.bfloat16)
            else:
                prev.wait_recv()
                prev.wait_send()
                rs_send[grp] = (
                    rs_in[grp, h - 1].astype(jnp.float32) + tile[grp, h % 2]
                ).astype(jnp.bfloat16)
            rdma = pltpu.make_async_remote_copy(
                src_ref=rs_send.at[grp],
                dst_ref=rs_in.at[grp, h],
                send_sem=rs_send_sems.at[grp, h],
                recv_sem=rs_recv_sems.at[grp, h],
                device_id=(my_x, y_right, my_z),
                device_id_type=pl.DeviceIdType.MESH,
            )
            rdma.start()
            load_start(grp, (h + 1) % 2,
                       rs_chunk(h + 1) if h < N_Y - 2 else my_y)
            return rdma

        def rs_finish(grp, prev):
            prev.wait_recv()
            prev.wait_send()
            load_wait(grp, (N_Y - 1) % 2)
            y_mine = (rs_in[grp, N_Y - 2].astype(jnp.float32)
                      + tile[grp, (N_Y - 1) % 2])
            y_all[:, pl.ds(col0 + grp * GW, GW)] = y_mine.astype(jnp.bfloat16)
            s = jnp.sum(y_mine * y_mine, axis=-1, keepdims=True)
            if grp == 0:
                ssq[...] = s
            else:
                ssq[...] = ssq[...] + s

        def ag_start(grp, k):
            o_cw = lax.rem(p + (N_P - k), N_P)
            o_ccw = lax.rem(p + k, N_P)
            c_cw = col_of_pos(o_cw) + grp * GW
            c_ccw = col_of_pos(o_ccw) + grp * GW
            Mh = M_BLK // 2
            cw = pltpu.make_async_remote_copy(
                src_ref=y_all.at[pl.ds(0, Mh), pl.ds(c_cw, GW)],
                dst_ref=y_all.at[pl.ds(0, Mh), pl.ds(c_cw, GW)],
                send_sem=ag_send_sems.at[grp, k, 0],
                recv_sem=ag_recv_sems.at[grp, k, 0],
                device_id=(nx, my_y, nz),
                device_id_type=pl.DeviceIdType.MESH,
            )
            ccw = pltpu.make_async_remote_copy(
                src_ref=y_all.at[pl.ds(Mh, Mh), pl.ds(c_ccw, GW)],
                dst_ref=y_all.at[pl.ds(Mh, Mh), pl.ds(c_ccw, GW)],
                send_sem=ag_send_sems.at[grp, k, 1],
                recv_sem=ag_recv_sems.at[grp, k, 1],
                device_id=(px, my_y, pz),
                device_id_type=pl.DeviceIdType.MESH,
            )
            cw.start()
            ccw.start()
            return cw, ccw

        def ag_wait(pair):
            pair[0].wait()
            pair[1].wait()

        def ssq_accum(grp, d):
            o = lax.rem(p + (N_P - d), N_P)
            oc = col_of_pos(o) + grp * GW
            v = y_all[:, pl.ds(oc, GW)].astype(jnp.float32)
            ssq[...] = ssq[...] + jnp.sum(v * v, axis=-1, keepdims=True)

        arrivals = {k: [d for d in range(1, N_P)
                        if max(d, N_P - d) - 1 == k]
                    for k in range(N_P - 1)}

        load_start(0, 0, rs_chunk(0))
        prev0 = None
        for h in range(N_Y - 1):
            prev0 = rs_hop(0, h, prev0)
            if h == 0:
                load_start(1, 0, rs_chunk(0))
        rs_finish(0, prev0)

        ag0 = {}
        ag1 = {}
        prev1 = None
        for k in range(N_P - 1):
            ag0[k] = ag_start(0, k)
            if k <= N_Y - 2:
                prev1 = rs_hop(1, k, prev1)
            elif k == N_Y - 1:
                rs_finish(1, prev1)
            else:
                ag1[k - N_Y] = ag_start(1, k - N_Y)
                for d in arrivals[k - 1]:
                    ssq_accum(0, d)
            ag_wait(ag0[k])
            if k >= N_Y:
                ag_wait(ag1[k - N_Y])

        for k in range(N_P - 1 - N_Y, N_P - 1):
            ag1[k] = ag_start(1, k)
            if k == N_P - 1 - N_Y:
                for d in arrivals[N_P - 2]:
                    ssq_accum(0, d)
            else:
                for d in arrivals[k - 1]:
                    ssq_accum(1, d)
            ag_wait(ag1[k])
        for d in arrivals[N_P - 2]:
            ssq_accum(1, d)

        rstd = lax.rsqrt(ssq[...] * (1.0 / D) + 1e-6)
        for t in range(M_BLK // T):
            sl = pl.ds(t * T, T)
            slot = t % 2
            if t >= 2:
                pltpu.make_async_copy(
                    out_stage.at[slot], out_ref.at[sl, :],
                    store_sems.at[slot]).wait()
            out_stage[slot] = (
                y_all[sl, :].astype(jnp.float32)
                * rstd[t * T:(t + 1) * T] * g_ref[...])
            pltpu.make_async_copy(
                out_stage.at[slot], out_ref.at[sl, :],
                store_sems.at[slot]).start()
        for slot in range(2):
            pltpu.make_async_copy(
                out_stage.at[slot], out_ref.at[pl.ds(0, T), :],
                store_sems.at[slot]).wait()

    return pl.pallas_call(
        body,
        out_shape=jax.ShapeDtypeStruct((M_BLK, D), jnp.float32),
        in_specs=[
            pl.BlockSpec(memory_space=pl.ANY),
            pl.BlockSpec(memory_space=pltpu.MemorySpace.VMEM),
        ],
        out_specs=pl.BlockSpec(memory_space=pl.ANY),
        scratch_shapes=[
            pltpu.VMEM((2, M_BLK, GW), jnp.bfloat16),
            pltpu.VMEM((2, N_Y - 1, M_BLK, GW), jnp.bfloat16),
            pltpu.VMEM((2, 2, M_BLK, GW), jnp.float32),
            pltpu.VMEM((M_BLK, D), jnp.bfloat16),
            pltpu.VMEM((M_BLK, 1), jnp.float32),
            pltpu.VMEM((2, T, D), jnp.float32),
            pltpu.SemaphoreType.DMA((2, 2)),
            pltpu.SemaphoreType.DMA((2,)),
            pltpu.SemaphoreType.DMA((2, N_Y - 1)),
            pltpu.SemaphoreType.DMA((2, N_Y - 1)),
            pltpu.SemaphoreType.DMA((2, N_P - 1, 2)),
            pltpu.SemaphoreType.DMA((2, N_P - 1, 2)),
        ],
        compiler_params=pltpu.CompilerParams(
            collective_id=0, vmem_limit_bytes=60 * 1024 * 1024),
    )(x, g)


# device time: 106149 ns/iter; 3.2377x vs baseline; 1.0770x over previous
import jax
import jax.numpy as jnp
from jax import lax
from jax.experimental import pallas as pl
from jax.experimental.pallas import tpu as pltpu

N_Y = 4
N_P = 8
M_BLK = 2048
D = 2048
W = D // N_P
GW = W // 2
GH = GW // 2
T = 512


def kernel(partial, gamma):
    x = partial.reshape(N_Y * M_BLK, D)
    g = gamma.reshape(1, D)

    def body(x_ref, g_ref, out_ref, rs_send, rs_in, tile, y_all, ssq,
             out_stage, load_sems, store_sems, rs_send_sems, rs_recv_sems,
             ag_send_sems, ag_recv_sems):
        my_x = lax.axis_index("x")
        my_y = lax.axis_index("y")
        my_z = lax.axis_index("z")
        y_left = lax.rem(my_y + (N_Y - 1), N_Y)
        y_right = lax.rem(my_y + 1, N_Y)

        p = jnp.where(my_x == 0, my_z, N_P - 1 - my_z)

        def pos_xz(o):
            return jnp.where(o < 4, 0, 1), jnp.where(o < 4, o, N_P - 1 - o)

        def col_of_pos(o):
            ox, oz = pos_xz(o)
            return (ox * 4 + oz) * W

        p_next = lax.rem(p + 1, N_P)
        p_prev = lax.rem(p + (N_P - 1), N_P)
        nx, nz = pos_xz(p_next)
        px, pz = pos_xz(p_prev)
        col0 = (my_x * 4 + my_z) * W

        barrier_sem = pltpu.get_barrier_semaphore()
        for dev in ((my_x, y_left, my_z), (my_x, y_right, my_z),
                    (nx, my_y, nz), (px, my_y, pz)):
            pl.semaphore_signal(
                barrier_sem, inc=1,
                device_id=dev, device_id_type=pl.DeviceIdType.MESH,
            )
        pl.semaphore_wait(barrier_sem, 4)

        def rs_chunk(h):
            return lax.rem(my_y + (N_Y - 1 - h), N_Y)

        def load_start(grp, slot, c):
            pltpu.make_async_copy(
                x_ref.at[pl.ds(c * M_BLK, M_BLK),
                         pl.ds(col0 + grp * GW, GW)],
                tile.at[grp, slot], load_sems.at[grp, slot]).start()

        def load_wait(grp, slot):
            pltpu.make_async_copy(
                x_ref.at[pl.ds(0, M_BLK), pl.ds(col0, GW)],
                tile.at[grp, slot], load_sems.at[grp, slot]).wait()

        Mh = M_BLK // 2

        def rs_sub(grp, h, half):
            rsl = pl.ds(half * Mh, Mh)
            return pltpu.make_async_remote_copy(
                src_ref=rs_send.at[grp, rsl],
                dst_ref=rs_in.at[grp, h, rsl],
                send_sem=rs_send_sems.at[grp, h, half],
                recv_sem=rs_recv_sems.at[grp, h, half],
                device_id=(my_x, y_right, my_z),
                device_id_type=pl.DeviceIdType.MESH,
            )

        def rs_hop(grp, h, prev):
            load_wait(grp, h % 2)
            subs = []
            for half in range(2):
                rsl = pl.ds(half * Mh, Mh)
                if h == 0:
                    rs_send[grp, rsl] = tile[grp, 0, rsl].astype(jnp.bfloat16)
                else:
                    prev[half].wait_recv()
                    prev[half].wait_send()
                    rs_send[grp, rsl] = (
                        rs_in[grp, h - 1, rsl].astype(jnp.float32)
                        + tile[grp, h % 2, rsl]
                    ).astype(jnp.bfloat16)
                sub = rs_sub(grp, h, half)
                sub.start()
                subs.append(sub)
            load_start(grp, (h + 1) % 2,
                       rs_chunk(h + 1) if h < N_Y - 2 else my_y)
            return subs

        def rs_finish(grp, prev):
            load_wait(grp, (N_Y - 1) % 2)
            for half in range(2):
                rsl = pl.ds(half * Mh, Mh)
                prev[half].wait_recv()
                prev[half].wait_send()
                y_mine = (rs_in[grp, N_Y - 2, rsl].astype(jnp.float32)
                          + tile[grp, (N_Y - 1) % 2, rsl])
                y_all[rsl, pl.ds(col0 + grp * GW, GW)] = (
                    y_mine.astype(jnp.bfloat16))
                s = jnp.sum(y_mine * y_mine, axis=-1, keepdims=True)
                if grp == 0:
                    ssq[rsl] = s
                else:
                    ssq[rsl] = ssq[rsl] + s

        def ag_start(grp, k):
            o_cw = lax.rem(p + (N_P - k), N_P)
            o_ccw = lax.rem(p + k, N_P)
            c_cw = col_of_pos(o_cw) + grp * GW
            c_ccw = col_of_pos(o_ccw) + grp * GW
            Mh = M_BLK // 2
            cw = pltpu.make_async_remote_copy(
                src_ref=y_all.at[pl.ds(0, Mh), pl.ds(c_cw, GW)],
                dst_ref=y_all.at[pl.ds(0, Mh), pl.ds(c_cw, GW)],
                send_sem=ag_send_sems.at[grp, k, 0],
                recv_sem=ag_recv_sems.at[grp, k, 0],
                device_id=(nx, my_y, nz),
                device_id_type=pl.DeviceIdType.MESH,
            )
            ccw = pltpu.make_async_remote_copy(
                src_ref=y_all.at[pl.ds(Mh, Mh), pl.ds(c_ccw, GW)],
                dst_ref=y_all.at[pl.ds(Mh, Mh), pl.ds(c_ccw, GW)],
                send_sem=ag_send_sems.at[grp, k, 1],
                recv_sem=ag_recv_sems.at[grp, k, 1],
                device_id=(px, my_y, pz),
                device_id_type=pl.DeviceIdType.MESH,
            )
            cw.start()
            ccw.start()
            return cw, ccw

        def ag_wait(pair):
            pair[0].wait()
            pair[1].wait()

        def ssq_accum(grp, d):
            o = lax.rem(p + (N_P - d), N_P)
            oc = col_of_pos(o) + grp * GW
            v = y_all[:, pl.ds(oc, GW)].astype(jnp.float32)
            ssq[...] = ssq[...] + jnp.sum(v * v, axis=-1, keepdims=True)

        arrivals = {k: [d for d in range(1, N_P)
                        if max(d, N_P - d) - 1 == k]
                    for k in range(N_P - 1)}

        load_start(0, 0, rs_chunk(0))
        prev0 = None
        for h in range(N_Y - 1):
            prev0 = rs_hop(0, h, prev0)
            if h == 0:
                load_start(1, 0, rs_chunk(0))
        rs_finish(0, prev0)

        ag0 = {}
        ag1 = {}
        prev1 = None
        for k in range(N_P - 1):
            ag0[k] = ag_start(0, k)
            if k <= N_Y - 2:
                prev1 = rs_hop(1, k, prev1)
            elif k == N_Y - 1:
                rs_finish(1, prev1)
            else:
                ag1[k - N_Y] = ag_start(1, k - N_Y)
                for d in arrivals[k - 1]:
                    ssq_accum(0, d)
            ag_wait(ag0[k])
            if k >= N_Y:
                ag_wait(ag1[k - N_Y])

        for k in range(N_P - 1 - N_Y, N_P - 1):
            ag1[k] = ag_start(1, k)
            if k == N_P - 1 - N_Y:
                for d in arrivals[N_P - 2]:
                    ssq_accum(0, d)
            else:
                for d in arrivals[k - 1]:
                    ssq_accum(1, d)
            ag_wait(ag1[k])
        for d in arrivals[N_P - 2]:
            ssq_accum(1, d)

        rstd = lax.rsqrt(ssq[...] * (1.0 / D) + 1e-6)
        for t in range(M_BLK // T):
            sl = pl.ds(t * T, T)
            slot = t % 2
            if t >= 2:
                pltpu.make_async_copy(
                    out_stage.at[slot], out_ref.at[sl, :],
                    store_sems.at[slot]).wait()
            out_stage[slot] = (
                y_all[sl, :].astype(jnp.float32)
                * rstd[t * T:(t + 1) * T] * g_ref[...])
            pltpu.make_async_copy(
                out_stage.at[slot], out_ref.at[sl, :],
                store_sems.at[slot]).start()
        for slot in range(2):
            pltpu.make_async_copy(
                out_stage.at[slot], out_ref.at[pl.ds(0, T), :],
                store_sems.at[slot]).wait()

    return pl.pallas_call(
        body,
        out_shape=jax.ShapeDtypeStruct((M_BLK, D), jnp.float32),
        in_specs=[
            pl.BlockSpec(memory_space=pl.ANY),
            pl.BlockSpec(memory_space=pltpu.MemorySpace.VMEM),
        ],
        out_specs=pl.BlockSpec(memory_space=pl.ANY),
        scratch_shapes=[
            pltpu.VMEM((2, M_BLK, GW), jnp.bfloat16),
            pltpu.VMEM((2, N_Y - 1, M_BLK, GW), jnp.bfloat16),
            pltpu.VMEM((2, 2, M_BLK, GW), jnp.float32),
            pltpu.VMEM((M_BLK, D), jnp.bfloat16),
            pltpu.VMEM((M_BLK, 1), jnp.float32),
            pltpu.VMEM((2, T, D), jnp.float32),
            pltpu.SemaphoreType.DMA((2, 2)),
            pltpu.SemaphoreType.DMA((2,)),
            pltpu.SemaphoreType.DMA((2, N_Y - 1, 2)),
            pltpu.SemaphoreType.DMA((2, N_Y - 1, 2)),
            pltpu.SemaphoreType.DMA((2, N_P - 1, 2)),
            pltpu.SemaphoreType.DMA((2, N_P - 1, 2)),
        ],
        compiler_params=pltpu.CompilerParams(
            collective_id=0, vmem_limit_bytes=60 * 1024 * 1024),
    )(x, g)


# device time: 101863 ns/iter; 3.3740x vs baseline; 1.0421x over previous
import jax
import jax.numpy as jnp
from jax import lax
from jax.experimental import pallas as pl
from jax.experimental.pallas import tpu as pltpu

N_Y = 4
N_P = 8
M_BLK = 2048
D = 2048
W = D // N_P
GW = W // 2
Mh = M_BLK // 2
Sh = Mh // 2
T = 512


def kernel(partial, gamma):
    x = partial.reshape(N_Y * M_BLK, D)
    g = gamma.reshape(1, D)

    def body(x_ref, g_ref, out_ref, rs_send, rs_in, tile, y_all, ssq,
             out_stage, load_sems, store_sems, rs_send_sems, rs_recv_sems,
             ag_send_sems, ag_recv_sems):
        my_x = lax.axis_index("x")
        my_y = lax.axis_index("y")
        my_z = lax.axis_index("z")
        y_left = lax.rem(my_y + (N_Y - 1), N_Y)
        y_right = lax.rem(my_y + 1, N_Y)

        p = jnp.where(my_x == 0, my_z, N_P - 1 - my_z)

        def pos_xz(o):
            return jnp.where(o < 4, 0, 1), jnp.where(o < 4, o, N_P - 1 - o)

        def col_of_pos(o):
            ox, oz = pos_xz(o)
            return (ox * 4 + oz) * W

        p_next = lax.rem(p + 1, N_P)
        p_prev = lax.rem(p + (N_P - 1), N_P)
        nx, nz = pos_xz(p_next)
        px, pz = pos_xz(p_prev)
        col0 = (my_x * 4 + my_z) * W

        def rs_chunk(h):
            return lax.rem(my_y + (N_Y - 1 - h), N_Y)

        def load_start(grp, slot, c):
            pltpu.make_async_copy(
                x_ref.at[pl.ds(c * M_BLK + grp * Mh, Mh), pl.ds(col0, W)],
                tile.at[grp, slot], load_sems.at[grp, slot]).start()

        def load_wait(grp, slot):
            pltpu.make_async_copy(
                x_ref.at[pl.ds(0, Mh), pl.ds(col0, W)],
                tile.at[grp, slot], load_sems.at[grp, slot]).wait()

        load_start(0, 0, rs_chunk(0))
        load_start(1, 0, rs_chunk(0))

        barrier_sem = pltpu.get_barrier_semaphore()
        for dev in ((my_x, y_left, my_z), (my_x, y_right, my_z),
                    (nx, my_y, nz), (px, my_y, pz)):
            pl.semaphore_signal(
                barrier_sem, inc=1,
                device_id=dev, device_id_type=pl.DeviceIdType.MESH,
            )
        pl.semaphore_wait(barrier_sem, 4)

        def rs_sub(grp, h, half):
            rsl = pl.ds(half * Sh, Sh)
            return pltpu.make_async_remote_copy(
                src_ref=rs_send.at[grp, rsl],
                dst_ref=rs_in.at[grp, h, rsl],
                send_sem=rs_send_sems.at[grp, h, half],
                recv_sem=rs_recv_sems.at[grp, h, half],
                device_id=(my_x, y_right, my_z),
                device_id_type=pl.DeviceIdType.MESH,
            )

        def rs_hop(grp, h, prev):
            load_wait(grp, h % 2)
            subs = []
            for half in range(2):
                rsl = pl.ds(half * Sh, Sh)
                if h == 0:
                    rs_send[grp, rsl] = tile[grp, 0, rsl].astype(jnp.bfloat16)
                else:
                    prev[half].wait_recv()
                    prev[half].wait_send()
                    rs_send[grp, rsl] = (
                        rs_in[grp, h - 1, rsl].astype(jnp.float32)
                        + tile[grp, h % 2, rsl]
                    ).astype(jnp.bfloat16)
                sub = rs_sub(grp, h, half)
                sub.start()
                subs.append(sub)
            load_start(grp, (h + 1) % 2,
                       rs_chunk(h + 1) if h < N_Y - 2 else my_y)
            return subs

        def rs_finish(grp, prev):
            load_wait(grp, (N_Y - 1) % 2)
            for half in range(2):
                rsl = pl.ds(half * Sh, Sh)
                prev[half].wait_recv()
                prev[half].wait_send()
                y_mine = (rs_in[grp, N_Y - 2, rsl].astype(jnp.float32)
                          + tile[grp, (N_Y - 1) % 2, rsl])
                y_all[pl.ds(grp * Mh + half * Sh, Sh), pl.ds(col0, W)] = (
                    y_mine.astype(jnp.bfloat16))
                ssq[pl.ds(grp * Mh + half * Sh, Sh)] = jnp.sum(
                    y_mine * y_mine, axis=-1, keepdims=True)

        def ag_start(grp, k):
            o_cw = lax.rem(p + (N_P - k), N_P)
            o_ccw = lax.rem(p + k, N_P)
            c_cw = col_of_pos(o_cw)
            c_ccw = col_of_pos(o_ccw) + GW
            rsl = pl.ds(grp * Mh, Mh)
            cw = pltpu.make_async_remote_copy(
                src_ref=y_all.at[rsl, pl.ds(c_cw, GW)],
                dst_ref=y_all.at[rsl, pl.ds(c_cw, GW)],
                send_sem=ag_send_sems.at[grp, k, 0],
                recv_sem=ag_recv_sems.at[grp, k, 0],
                device_id=(nx, my_y, nz),
                device_id_type=pl.DeviceIdType.MESH,
            )
            ccw = pltpu.make_async_remote_copy(
                src_ref=y_all.at[rsl, pl.ds(c_ccw, GW)],
                dst_ref=y_all.at[rsl, pl.ds(c_ccw, GW)],
                send_sem=ag_send_sems.at[grp, k, 1],
                recv_sem=ag_recv_sems.at[grp, k, 1],
                device_id=(px, my_y, pz),
                device_id_type=pl.DeviceIdType.MESH,
            )
            cw.start()
            ccw.start()
            return cw, ccw

        def ag_wait(pair):
            pair[0].wait()
            pair[1].wait()

        def ssq_accum(grp, d):
            o = lax.rem(p + (N_P - d), N_P)
            oc = col_of_pos(o)
            rsl = pl.ds(grp * Mh, Mh)
            v = y_all[rsl, pl.ds(oc, W)].astype(jnp.float32)
            ssq[rsl] = ssq[rsl] + jnp.sum(v * v, axis=-1, keepdims=True)

        arrivals = {k: [d for d in range(1, N_P)
                        if max(d, N_P - d) - 1 == k]
                    for k in range(N_P - 1)}

        pending = [False, False]

        def norm_tile(t):
            slot = t % 2
            if pending[slot]:
                pltpu.make_async_copy(
                    out_stage.at[slot], out_ref.at[pl.ds(0, T), :],
                    store_sems.at[slot]).wait()
            rsl = pl.ds(t * T, T)
            rstd = lax.rsqrt(ssq[rsl] * (1.0 / D) + 1e-6)
            out_stage[slot] = y_all[rsl, :].astype(jnp.float32) * rstd * g_ref[...]
            pltpu.make_async_copy(
                out_stage.at[slot], out_ref.at[rsl, :],
                store_sems.at[slot]).start()
            pending[slot] = True

        prev0 = None
        for h in range(N_Y - 1):
            prev0 = rs_hop(0, h, prev0)
        rs_finish(0, prev0)

        ag0 = {}
        ag1 = {}
        prev1 = None
        for k in range(N_P - 1):
            ag0[k] = ag_start(0, k)
            if k <= N_Y - 2:
                prev1 = rs_hop(1, k, prev1)
            elif k == N_Y - 1:
                rs_finish(1, prev1)
            else:
                ag1[k - N_Y] = ag_start(1, k - N_Y)
                for d in arrivals[k - 1]:
                    ssq_accum(0, d)
            ag_wait(ag0[k])
            if k >= N_Y:
                ag_wait(ag1[k - N_Y])

        for k in range(N_P - 1 - N_Y, N_P - 1):
            ag1[k] = ag_start(1, k)
            if k == 3:
                for d in arrivals[N_P - 2]:
                    ssq_accum(0, d)
            elif k == 4:
                for d in arrivals[3]:
                    ssq_accum(1, d)
                norm_tile(0)
            elif k == 5:
                for d in arrivals[4]:
                    ssq_accum(1, d)
                norm_tile(1)
            else:
                for d in arrivals[5]:
                    ssq_accum(1, d)
            ag_wait(ag1[k])
        for d in arrivals[N_P - 2]:
            ssq_accum(1, d)
        norm_tile(2)
        norm_tile(3)
        for slot in range(2):
            pltpu.make_async_copy(
                out_stage.at[slot], out_ref.at[pl.ds(0, T), :],
                store_sems.at[slot]).wait()

    return pl.pallas_call(
        body,
        out_shape=jax.ShapeDtypeStruct((M_BLK, D), jnp.float32),
        in_specs=[
            pl.BlockSpec(memory_space=pl.ANY),
            pl.BlockSpec(memory_space=pltpu.MemorySpace.VMEM),
        ],
        out_specs=pl.BlockSpec(memory_space=pl.ANY),
        scratch_shapes=[
            pltpu.VMEM((2, Mh, W), jnp.bfloat16),
            pltpu.VMEM((2, N_Y - 1, Mh, W), jnp.bfloat16),
            pltpu.VMEM((2, 2, Mh, W), jnp.float32),
            pltpu.VMEM((M_BLK, D), jnp.bfloat16),
            pltpu.VMEM((M_BLK, 1), jnp.float32),
            pltpu.VMEM((2, T, D), jnp.float32),
            pltpu.SemaphoreType.DMA((2, 2)),
            pltpu.SemaphoreType.DMA((2,)),
            pltpu.SemaphoreType.DMA((2, N_Y - 1, 2)),
            pltpu.SemaphoreType.DMA((2, N_Y - 1, 2)),
            pltpu.SemaphoreType.DMA((2, N_P - 1, 2)),
            pltpu.SemaphoreType.DMA((2, N_P - 1, 2)),
        ],
        compiler_params=pltpu.CompilerParams(
            collective_id=0, vmem_limit_bytes=60 * 1024 * 1024),
    )(x, g)


# device time: 93674 ns/iter; 3.6689x vs baseline; 1.0874x over previous
import jax
import jax.numpy as jnp
from jax import lax
from jax.experimental import pallas as pl
from jax.experimental.pallas import tpu as pltpu

N_Y = 4
N_P = 8
M_BLK = 2048
D = 2048
W = D // N_P
GW = W // 2
Mh = M_BLK // 2
RS_SUBS = 4
Sh = Mh // RS_SUBS
AH = Mh // 2
T = 512


def kernel(partial, gamma):
    x = partial.reshape(N_Y * M_BLK, D)
    g = gamma.reshape(1, D)

    def body(x_ref, g_ref, out_ref, rs_send, rs_in, tile, y_all, ssq,
             out_stage, load_sems, store_sems, rs_send_sems, rs_recv_sems,
             ag_send_sems, ag_recv_sems):
        my_x = lax.axis_index("x")
        my_y = lax.axis_index("y")
        my_z = lax.axis_index("z")
        y_left = lax.rem(my_y + (N_Y - 1), N_Y)
        y_right = lax.rem(my_y + 1, N_Y)

        p = jnp.where(my_x == 0, my_z, N_P - 1 - my_z)

        def pos_xz(o):
            return jnp.where(o < 4, 0, 1), jnp.where(o < 4, o, N_P - 1 - o)

        def col_of_pos(o):
            ox, oz = pos_xz(o)
            return (ox * 4 + oz) * W

        p_next = lax.rem(p + 1, N_P)
        p_prev = lax.rem(p + (N_P - 1), N_P)
        nx, nz = pos_xz(p_next)
        px, pz = pos_xz(p_prev)
        col0 = (my_x * 4 + my_z) * W

        def rs_chunk(h):
            return lax.rem(my_y + (N_Y - 1 - h), N_Y)

        def load_start(grp, slot, c):
            pltpu.make_async_copy(
                x_ref.at[pl.ds(c * M_BLK + grp * Mh, Mh), pl.ds(col0, W)],
                tile.at[grp, slot], load_sems.at[grp, slot]).start()

        def load_wait(grp, slot):
            pltpu.make_async_copy(
                x_ref.at[pl.ds(0, Mh), pl.ds(col0, W)],
                tile.at[grp, slot], load_sems.at[grp, slot]).wait()

        load_start(0, 0, rs_chunk(0))
        load_start(1, 0, rs_chunk(0))

        barrier_sem = pltpu.get_barrier_semaphore()
        for dev in ((my_x, y_left, my_z), (my_x, y_right, my_z),
                    (nx, my_y, nz), (px, my_y, pz)):
            pl.semaphore_signal(
                barrier_sem, inc=1,
                device_id=dev, device_id_type=pl.DeviceIdType.MESH,
            )
        pl.semaphore_wait(barrier_sem, 4)

        def rs_sub(grp, h, half):
            rsl = pl.ds(half * Sh, Sh)
            return pltpu.make_async_remote_copy(
                src_ref=rs_send.at[grp, rsl],
                dst_ref=rs_in.at[grp, h, rsl],
                send_sem=rs_send_sems.at[grp, h, half],
                recv_sem=rs_recv_sems.at[grp, h, half],
                device_id=(my_x, y_right, my_z),
                device_id_type=pl.DeviceIdType.MESH,
            )

        def rs_hop(grp, h, prev):
            load_wait(grp, h % 2)
            subs = []
            for half in range(RS_SUBS):
                rsl = pl.ds(half * Sh, Sh)
                if h == 0:
                    rs_send[grp, rsl] = tile[grp, 0, rsl].astype(jnp.bfloat16)
                else:
                    prev[half].wait_recv()
                    prev[half].wait_send()
                    rs_send[grp, rsl] = (
                        rs_in[grp, h - 1, rsl].astype(jnp.float32)
                        + tile[grp, h % 2, rsl]
                    ).astype(jnp.bfloat16)
                sub = rs_sub(grp, h, half)
                sub.start()
                subs.append(sub)
            load_start(grp, (h + 1) % 2,
                       rs_chunk(h + 1) if h < N_Y - 2 else my_y)
            return subs

        def rs_finish(grp, prev):
            load_wait(grp, (N_Y - 1) % 2)
            for half in range(RS_SUBS):
                rsl = pl.ds(half * Sh, Sh)
                prev[half].wait_recv()
                prev[half].wait_send()
                y_mine = (rs_in[grp, N_Y - 2, rsl].astype(jnp.float32)
                          + tile[grp, (N_Y - 1) % 2, rsl])
                y_all[pl.ds(grp * Mh + half * Sh, Sh), pl.ds(col0, W)] = (
                    y_mine.astype(jnp.bfloat16))
                ssq[pl.ds(grp * Mh + half * Sh, Sh)] = jnp.sum(
                    y_mine * y_mine, axis=-1, keepdims=True)

        def ag_subs(grp, k):
            o_cw = lax.rem(p + (N_P - k), N_P)
            o_ccw = lax.rem(p + k, N_P)
            descs = []
            for sub in range(2):
                rsl = pl.ds(grp * Mh + sub * AH, AH)
                for dir_idx, oc, tgt in (
                        (0, col_of_pos(o_cw), (nx, my_y, nz)),
                        (1, col_of_pos(o_ccw) + GW, (px, my_y, pz))):
                    descs.append(pltpu.make_async_remote_copy(
                        src_ref=y_all.at[rsl, pl.ds(oc, GW)],
                        dst_ref=y_all.at[rsl, pl.ds(oc, GW)],
                        send_sem=ag_send_sems.at[grp, k, dir_idx, sub],
                        recv_sem=ag_recv_sems.at[grp, k, dir_idx, sub],
                        device_id=tgt,
                        device_id_type=pl.DeviceIdType.MESH,
                    ))
            return descs

        def ag_stage(grp, k, prev):
            cur = ag_subs(grp, k)
            for i in range(4):
                if prev is not None:
                    prev[i].wait_recv()
                    prev[i].wait_send()
                cur[i].start()
            return cur

        def ag_last_wait(prev):
            for i in range(4):
                prev[i].wait_recv()
                prev[i].wait_send()

        def ssq_accum(grp, d):
            o = lax.rem(p + (N_P - d), N_P)
            oc = col_of_pos(o)
            rsl = pl.ds(grp * Mh, Mh)
            v = y_all[rsl, pl.ds(oc, W)].astype(jnp.float32)
            ssq[rsl] = ssq[rsl] + jnp.sum(v * v, axis=-1, keepdims=True)

        arrivals = {k: [d for d in range(1, N_P)
                        if max(d, N_P - d) - 1 == k]
                    for k in range(N_P - 1)}

        pending = [False, False]

        def norm_tile(t):
            slot = t % 2
            if pending[slot]:
                pltpu.make_async_copy(
                    out_stage.at[slot], out_ref.at[pl.ds(0, T), :],
                    store_sems.at[slot]).wait()
            rsl = pl.ds(t * T, T)
            rstd = lax.rsqrt(ssq[rsl] * (1.0 / D) + 1e-6)
            out_stage[slot] = y_all[rsl, :].astype(jnp.float32) * rstd * g_ref[...]
            pltpu.make_async_copy(
                out_stage.at[slot], out_ref.at[rsl, :],
                store_sems.at[slot]).start()
            pending[slot] = True

        prev0 = None
        for h in range(N_Y - 1):
            prev0 = rs_hop(0, h, prev0)
        rs_finish(0, prev0)

        ag0_prev = None
        ag1_prev = None
        prev1 = None
        for k in range(N_P - 1):
            ag0_prev = ag_stage(0, k, ag0_prev)
            if k <= N_Y - 2:
                prev1 = rs_hop(1, k, prev1)
            elif k == N_Y - 1:
                rs_finish(1, prev1)
            else:
                ag1_prev = ag_stage(1, k - N_Y, ag1_prev)
                for d in arrivals[k - 1]:
                    ssq_accum(0, d)

        for k in range(N_P - 1 - N_Y, N_P - 1):
            ag1_prev = ag_stage(1, k, ag1_prev)
            if k == 3:
                ag_last_wait(ag0_prev)
                for d in arrivals[N_P - 2]:
                    ssq_accum(0, d)
            elif k == 4:
                for d in arrivals[3]:
                    ssq_accum(1, d)
                norm_tile(0)
            elif k == 5:
                for d in arrivals[4]:
                    ssq_accum(1, d)
                norm_tile(1)
            else:
                for d in arrivals[5]:
                    ssq_accum(1, d)
        ag_last_wait(ag1_prev)
        for d in arrivals[N_P - 2]:
            ssq_accum(1, d)
        norm_tile(2)
        norm_tile(3)
        for slot in range(2):
            pltpu.make_async_copy(
                out_stage.at[slot], out_ref.at[pl.ds(0, T), :],
                store_sems.at[slot]).wait()

    return pl.pallas_call(
        body,
        out_shape=jax.ShapeDtypeStruct((M_BLK, D), jnp.float32),
        in_specs=[
            pl.BlockSpec(memory_space=pl.ANY),
            pl.BlockSpec(memory_space=pltpu.MemorySpace.VMEM),
        ],
        out_specs=pl.BlockSpec(memory_space=pl.ANY),
        scratch_shapes=[
            pltpu.VMEM((2, Mh, W), jnp.bfloat16),
            pltpu.VMEM((2, N_Y - 1, Mh, W), jnp.bfloat16),
            pltpu.VMEM((2, 2, Mh, W), jnp.float32),
            pltpu.VMEM((M_BLK, D), jnp.bfloat16),
            pltpu.VMEM((M_BLK, 1), jnp.float32),
            pltpu.VMEM((2, T, D), jnp.float32),
            pltpu.SemaphoreType.DMA((2, 2)),
            pltpu.SemaphoreType.DMA((2,)),
            pltpu.SemaphoreType.DMA((2, N_Y - 1, RS_SUBS)),
            pltpu.SemaphoreType.DMA((2, N_Y - 1, RS_SUBS)),
            pltpu.SemaphoreType.DMA((2, N_P - 1, 2, 2)),
            pltpu.SemaphoreType.DMA((2, N_P - 1, 2, 2)),
        ],
        compiler_params=pltpu.CompilerParams(
            collective_id=0, vmem_limit_bytes=60 * 1024 * 1024),
    )(x, g)


# device time: 93140 ns/iter; 3.6900x vs baseline; 1.0057x over previous
import jax
import jax.numpy as jnp
from jax import lax
from jax.experimental import pallas as pl
from jax.experimental.pallas import tpu as pltpu

N_Y = 4
N_P = 8
M_BLK = 2048
D = 2048
W = D // N_P
GW = W // 2
Mh = M_BLK // 2
RS_SUBS = 8
Sh = Mh // RS_SUBS
AG_SUBS = 4
AH = Mh // AG_SUBS
T = 512


def kernel(partial, gamma):
    x = partial.reshape(N_Y * M_BLK, D)
    g = gamma.reshape(1, D)

    def body(x_ref, g_ref, out_ref, rs_send, rs_in, tile, y_all, ssq,
             out_stage, load_sems, store_sems, rs_send_sems, rs_recv_sems,
             ag_send_sems, ag_recv_sems):
        my_x = lax.axis_index("x")
        my_y = lax.axis_index("y")
        my_z = lax.axis_index("z")
        y_left = lax.rem(my_y + (N_Y - 1), N_Y)
        y_right = lax.rem(my_y + 1, N_Y)

        p = jnp.where(my_x == 0, my_z, N_P - 1 - my_z)

        def pos_xz(o):
            return jnp.where(o < 4, 0, 1), jnp.where(o < 4, o, N_P - 1 - o)

        def col_of_pos(o):
            ox, oz = pos_xz(o)
            return (ox * 4 + oz) * W

        p_next = lax.rem(p + 1, N_P)
        p_prev = lax.rem(p + (N_P - 1), N_P)
        nx, nz = pos_xz(p_next)
        px, pz = pos_xz(p_prev)
        col0 = (my_x * 4 + my_z) * W

        def rs_chunk(h):
            return lax.rem(my_y + (N_Y - 1 - h), N_Y)

        def load_start(grp, slot, c):
            pltpu.make_async_copy(
                x_ref.at[pl.ds(c * M_BLK + grp * Mh, Mh), pl.ds(col0, W)],
                tile.at[grp, slot], load_sems.at[grp, slot]).start()

        def load_wait(grp, slot):
            pltpu.make_async_copy(
                x_ref.at[pl.ds(0, Mh), pl.ds(col0, W)],
                tile.at[grp, slot], load_sems.at[grp, slot]).wait()

        load_start(0, 0, rs_chunk(0))
        load_start(1, 0, rs_chunk(0))

        barrier_sem = pltpu.get_barrier_semaphore()
        for dev in ((my_x, y_left, my_z), (my_x, y_right, my_z),
                    (nx, my_y, nz), (px, my_y, pz)):
            pl.semaphore_signal(
                barrier_sem, inc=1,
                device_id=dev, device_id_type=pl.DeviceIdType.MESH,
            )
        pl.semaphore_wait(barrier_sem, 4)

        def rs_sub(grp, h, half):
            rsl = pl.ds(half * Sh, Sh)
            return pltpu.make_async_remote_copy(
                src_ref=rs_send.at[grp, rsl],
                dst_ref=rs_in.at[grp, h, rsl],
                send_sem=rs_send_sems.at[grp, h, half],
                recv_sem=rs_recv_sems.at[grp, h, half],
                device_id=(my_x, y_right, my_z),
                device_id_type=pl.DeviceIdType.MESH,
            )

        def rs_hop(grp, h, prev):
            load_wait(grp, h % 2)
            subs = []
            for half in range(RS_SUBS):
                rsl = pl.ds(half * Sh, Sh)
                if h == 0:
                    rs_send[grp, rsl] = tile[grp, 0, rsl].astype(jnp.bfloat16)
                else:
                    prev[half].wait_recv()
                    prev[half].wait_send()
                    rs_send[grp, rsl] = (
                        rs_in[grp, h - 1, rsl].astype(jnp.float32)
                        + tile[grp, h % 2, rsl]
                    ).astype(jnp.bfloat16)
                sub = rs_sub(grp, h, half)
                sub.start()
                subs.append(sub)
            load_start(grp, (h + 1) % 2,
                       rs_chunk(h + 1) if h < N_Y - 2 else my_y)
            return subs

        def rs_finish(grp, prev):
            load_wait(grp, (N_Y - 1) % 2)
            for half in range(RS_SUBS):
                rsl = pl.ds(half * Sh, Sh)
                prev[half].wait_recv()
                prev[half].wait_send()
                y_mine = (rs_in[grp, N_Y - 2, rsl].astype(jnp.float32)
                          + tile[grp, (N_Y - 1) % 2, rsl])
                y_all[pl.ds(grp * Mh + half * Sh, Sh), pl.ds(col0, W)] = (
                    y_mine.astype(jnp.bfloat16))
                ssq[pl.ds(grp * Mh + half * Sh, Sh)] = jnp.sum(
                    y_mine * y_mine, axis=-1, keepdims=True)

        def ag_subs(grp, k):
            o_cw = lax.rem(p + (N_P - k), N_P)
            o_ccw = lax.rem(p + k, N_P)
            descs = []
            for sub in range(AG_SUBS):
                rsl = pl.ds(grp * Mh + sub * AH, AH)
                for dir_idx, oc, tgt in (
                        (0, col_of_pos(o_cw), (nx, my_y, nz)),
                        (1, col_of_pos(o_ccw) + GW, (px, my_y, pz))):
                    descs.append(pltpu.make_async_remote_copy(
                        src_ref=y_all.at[rsl, pl.ds(oc, GW)],
                        dst_ref=y_all.at[rsl, pl.ds(oc, GW)],
                        send_sem=ag_send_sems.at[grp, k, dir_idx, sub],
                        recv_sem=ag_recv_sems.at[grp, k, dir_idx, sub],
                        device_id=tgt,
                        device_id_type=pl.DeviceIdType.MESH,
                    ))
            return descs

        def ag_stage(grp, k, prev):
            cur = ag_subs(grp, k)
            for i in range(2 * AG_SUBS):
                if prev is not None:
                    prev[i].wait_recv()
                    prev[i].wait_send()
                cur[i].start()
            return cur

        def ag_last_wait(prev):
            for i in range(2 * AG_SUBS):
                prev[i].wait_recv()
                prev[i].wait_send()

        def ssq_accum(grp, d):
            o = lax.rem(p + (N_P - d), N_P)
            oc = col_of_pos(o)
            rsl = pl.ds(grp * Mh, Mh)
            v = y_all[rsl, pl.ds(oc, W)].astype(jnp.float32)
            ssq[rsl] = ssq[rsl] + jnp.sum(v * v, axis=-1, keepdims=True)

        arrivals = {k: [d for d in range(1, N_P)
                        if max(d, N_P - d) - 1 == k]
                    for k in range(N_P - 1)}

        pending = [False, False]

        def norm_tile(t):
            slot = t % 2
            if pending[slot]:
                pltpu.make_async_copy(
                    out_stage.at[slot], out_ref.at[pl.ds(0, T), :],
                    store_sems.at[slot]).wait()
            rsl = pl.ds(t * T, T)
            rstd = lax.rsqrt(ssq[rsl] * (1.0 / D) + 1e-6)
            out_stage[slot] = y_all[rsl, :].astype(jnp.float32) * rstd * g_ref[...]
            pltpu.make_async_copy(
                out_stage.at[slot], out_ref.at[rsl, :],
                store_sems.at[slot]).start()
            pending[slot] = True

        prev0 = None
        for h in range(N_Y - 1):
            prev0 = rs_hop(0, h, prev0)
        rs_finish(0, prev0)

        ag0_prev = None
        ag1_prev = None
        prev1 = None
        for k in range(N_P - 1):
            ag0_prev = ag_stage(0, k, ag0_prev)
            if k <= N_Y - 2:
                prev1 = rs_hop(1, k, prev1)
            elif k == N_Y - 1:
                rs_finish(1, prev1)
            else:
                ag1_prev = ag_stage(1, k - N_Y, ag1_prev)
                for d in arrivals[k - 1]:
                    ssq_accum(0, d)

        for k in range(N_P - 1 - N_Y, N_P - 1):
            ag1_prev = ag_stage(1, k, ag1_prev)
            if k == 3:
                ag_last_wait(ag0_prev)
                for d in arrivals[N_P - 2]:
                    ssq_accum(0, d)
            elif k == 4:
                for d in arrivals[3]:
                    ssq_accum(1, d)
                norm_tile(0)
            elif k == 5:
                for d in arrivals[4]:
                    ssq_accum(1, d)
                norm_tile(1)
            else:
                for d in arrivals[5]:
                    ssq_accum(1, d)
        ag_last_wait(ag1_prev)
        for d in arrivals[N_P - 2]:
            ssq_accum(1, d)
        norm_tile(2)
        norm_tile(3)
        for slot in range(2):
            pltpu.make_async_copy(
                out_stage.at[slot], out_ref.at[pl.ds(0, T), :],
                store_sems.at[slot]).wait()

    return pl.pallas_call(
        body,
        out_shape=jax.ShapeDtypeStruct((M_BLK, D), jnp.float32),
        in_specs=[
            pl.BlockSpec(memory_space=pl.ANY),
            pl.BlockSpec(memory_space=pltpu.MemorySpace.VMEM),
        ],
        out_specs=pl.BlockSpec(memory_space=pl.ANY),
        scratch_shapes=[
            pltpu.VMEM((2, Mh, W), jnp.bfloat16),
            pltpu.VMEM((2, N_Y - 1, Mh, W), jnp.bfloat16),
            pltpu.VMEM((2, 2, Mh, W), jnp.float32),
            pltpu.VMEM((M_BLK, D), jnp.bfloat16),
            pltpu.VMEM((M_BLK, 1), jnp.float32),
            pltpu.VMEM((2, T, D), jnp.float32),
            pltpu.SemaphoreType.DMA((2, 2)),
            pltpu.SemaphoreType.DMA((2,)),
            pltpu.SemaphoreType.DMA((2, N_Y - 1, RS_SUBS)),
            pltpu.SemaphoreType.DMA((2, N_Y - 1, RS_SUBS)),
            pltpu.SemaphoreType.DMA((2, N_P - 1, 2, AG_SUBS)),
            pltpu.SemaphoreType.DMA((2, N_P - 1, 2, AG_SUBS)),
        ],
        compiler_params=pltpu.CompilerParams(
            collective_id=0, vmem_limit_bytes=60 * 1024 * 1024),
    )(x, g)


# device time: 90041 ns/iter; 3.8170x vs baseline; 1.0344x over previous
import jax
import jax.numpy as jnp
from jax import lax
from jax.experimental import pallas as pl
from jax.experimental.pallas import tpu as pltpu

N_Y = 4
N_P = 8
N_CW = 4
N_CCW = 3
M_BLK = 2048
D = 2048
W = D // N_P
Mh = M_BLK // 2
RS_SUBS = 8
Sh = Mh // RS_SUBS
AG_SUBS = 2
AH = Mh // AG_SUBS
T = 512


def kernel(partial, gamma):
    x = partial.reshape(N_Y * M_BLK, D)
    g = gamma.reshape(1, D)

    def body(x_ref, g_ref, out_ref, rs_send, rs_in, tile, y_all,
             out_stage, load_sems, store_sems, rs_send_sems, rs_recv_sems,
             ag_send_sems, ag_recv_sems):
        my_x = lax.axis_index("x")
        my_y = lax.axis_index("y")
        my_z = lax.axis_index("z")
        y_left = lax.rem(my_y + (N_Y - 1), N_Y)
        y_right = lax.rem(my_y + 1, N_Y)

        p = jnp.where(my_x == 0, my_z, N_P - 1 - my_z)

        def pos_xz(o):
            return jnp.where(o < 4, 0, 1), jnp.where(o < 4, o, N_P - 1 - o)

        def col_of_pos(o):
            ox, oz = pos_xz(o)
            return (ox * 4 + oz) * W

        p_next = lax.rem(p + 1, N_P)
        p_prev = lax.rem(p + (N_P - 1), N_P)
        nx, nz = pos_xz(p_next)
        px, pz = pos_xz(p_prev)
        col0 = (my_x * 4 + my_z) * W

        def rs_chunk(h):
            return lax.rem(my_y + (N_Y - 1 - h), N_Y)

        def load_start(grp, slot, c):
            pltpu.make_async_copy(
                x_ref.at[pl.ds(c * M_BLK + grp * Mh, Mh), pl.ds(col0, W)],
                tile.at[grp, slot], load_sems.at[grp, slot]).start()

        def load_wait(grp, slot):
            pltpu.make_async_copy(
                x_ref.at[pl.ds(0, Mh), pl.ds(col0, W)],
                tile.at[grp, slot], load_sems.at[grp, slot]).wait()

        load_start(0, 0, rs_chunk(0))
        load_start(1, 0, rs_chunk(0))

        barrier_sem = pltpu.get_barrier_semaphore()
        for dev in ((my_x, y_left, my_z), (my_x, y_right, my_z),
                    (nx, my_y, nz), (px, my_y, pz)):
            pl.semaphore_signal(
                barrier_sem, inc=1,
                device_id=dev, device_id_type=pl.DeviceIdType.MESH,
            )
        pl.semaphore_wait(barrier_sem, 4)

        def rs_sub(grp, h, half):
            rsl = pl.ds(half * Sh, Sh)
            return pltpu.make_async_remote_copy(
                src_ref=rs_send.at[grp, rsl],
                dst_ref=rs_in.at[grp, h, rsl],
                send_sem=rs_send_sems.at[grp, h, half],
                recv_sem=rs_recv_sems.at[grp, h, half],
                device_id=(my_x, y_right, my_z),
                device_id_type=pl.DeviceIdType.MESH,
            )

        def rs_hop(grp, h, prev):
            load_wait(grp, h % 2)
            subs = []
            for half in range(RS_SUBS):
                rsl = pl.ds(half * Sh, Sh)
                if h == 0:
                    rs_send[grp, rsl] = tile[grp, 0, rsl].astype(jnp.bfloat16)
                else:
                    prev[half].wait_recv()
                    prev[half].wait_send()
                    rs_send[grp, rsl] = (
                        rs_in[grp, h - 1, rsl].astype(jnp.float32)
                        + tile[grp, h % 2, rsl]
                    ).astype(jnp.bfloat16)
                sub = rs_sub(grp, h, half)
                sub.start()
                subs.append(sub)
            load_start(grp, (h + 1) % 2,
                       rs_chunk(h + 1) if h < N_Y - 2 else my_y)
            return subs

        def rs_finish(grp, prev):
            load_wait(grp, (N_Y - 1) % 2)
            for half in range(RS_SUBS):
                rsl = pl.ds(half * Sh, Sh)
                prev[half].wait_recv()
                prev[half].wait_send()
                y_mine = (rs_in[grp, N_Y - 2, rsl].astype(jnp.float32)
                          + tile[grp, (N_Y - 1) % 2, rsl])
                y_all[pl.ds(grp * Mh + half * Sh, Sh), pl.ds(col0, W)] = (
                    y_mine.astype(jnp.bfloat16))

        def ag_desc(grp, k, dir_idx, sub):
            if dir_idx == 0:
                o = lax.rem(p + (N_P - k), N_P)
                tgt = (nx, my_y, nz)
            else:
                o = lax.rem(p + k, N_P)
                tgt = (px, my_y, pz)
            rsl = pl.ds(grp * Mh + sub * AH, AH)
            csl = pl.ds(col_of_pos(o), W)
            return pltpu.make_async_remote_copy(
                src_ref=y_all.at[rsl, csl],
                dst_ref=y_all.at[rsl, csl],
                send_sem=ag_send_sems.at[grp, k, dir_idx, sub],
                recv_sem=ag_recv_sems.at[grp, k, dir_idx, sub],
                device_id=tgt,
                device_id_type=pl.DeviceIdType.MESH,
            )

        def ag_stage(grp, k, prev):
            pcw, pccw = prev if prev is not None else (None, None)
            cw = [ag_desc(grp, k, 0, s) for s in range(AG_SUBS)]
            ccw = ([ag_desc(grp, k, 1, s) for s in range(AG_SUBS)]
                   if k < N_CCW else None)
            for s in range(AG_SUBS):
                if pcw is not None:
                    pcw[s].wait_recv()
                    pcw[s].wait_send()
                cw[s].start()
                if pccw is not None:
                    pccw[s].wait_recv()
                    pccw[s].wait_send()
                if ccw is not None:
                    ccw[s].start()
            return cw, ccw

        def ag_last_wait(prev):
            pcw, _ = prev
            for s in range(AG_SUBS):
                pcw[s].wait_recv()
                pcw[s].wait_send()

        pending = [False, False]

        def norm_tile(t):
            slot = t % 2
            if pending[slot]:
                pltpu.make_async_copy(
                    out_stage.at[slot], out_ref.at[pl.ds(0, T), :],
                    store_sems.at[slot]).wait()
            rsl = pl.ds(t * T, T)
            y = y_all[rsl, :].astype(jnp.float32)
            ms = jnp.mean(y * y, axis=-1, keepdims=True)
            out_stage[slot] = y * lax.rsqrt(ms + 1e-6) * g_ref[...]
            pltpu.make_async_copy(
                out_stage.at[slot], out_ref.at[rsl, :],
                store_sems.at[slot]).start()
            pending[slot] = True

        prev0 = None
        for h in range(N_Y - 1):
            prev0 = rs_hop(0, h, prev0)
        rs_finish(0, prev0)

        ag0 = None
        prev1 = None
        for k in range(N_CW):
            ag0 = ag_stage(0, k, ag0)
            if k <= N_Y - 2:
                prev1 = rs_hop(1, k, prev1)
            else:
                rs_finish(1, prev1)

        ag1 = None
        for k in range(N_CW):
            ag1 = ag_stage(1, k, ag1)
            if k == N_CW - 2:
                ag_last_wait(ag0)
                norm_tile(0)
            elif k == N_CW - 1:
                norm_tile(1)
        ag_last_wait(ag1)
        norm_tile(2)
        norm_tile(3)
        for slot in range(2):
            pltpu.make_async_copy(
                out_stage.at[slot], out_ref.at[pl.ds(0, T), :],
                store_sems.at[slot]).wait()

    return pl.pallas_call(
        body,
        out_shape=jax.ShapeDtypeStruct((M_BLK, D), jnp.float32),
        in_specs=[
            pl.BlockSpec(memory_space=pl.ANY),
            pl.BlockSpec(memory_space=pltpu.MemorySpace.VMEM),
        ],
        out_specs=pl.BlockSpec(memory_space=pl.ANY),
        scratch_shapes=[
            pltpu.VMEM((2, Mh, W), jnp.bfloat16),
            pltpu.VMEM((2, N_Y - 1, Mh, W), jnp.bfloat16),
            pltpu.VMEM((2, 2, Mh, W), jnp.float32),
            pltpu.VMEM((M_BLK, D), jnp.bfloat16),
            pltpu.VMEM((2, T, D), jnp.float32),
            pltpu.SemaphoreType.DMA((2, 2)),
            pltpu.SemaphoreType.DMA((2,)),
            pltpu.SemaphoreType.DMA((2, N_Y - 1, RS_SUBS)),
            pltpu.SemaphoreType.DMA((2, N_Y - 1, RS_SUBS)),
            pltpu.SemaphoreType.DMA((2, N_CW, 2, AG_SUBS)),
            pltpu.SemaphoreType.DMA((2, N_CW, 2, AG_SUBS)),
        ],
        compiler_params=pltpu.CompilerParams(
            collective_id=0, vmem_limit_bytes=60 * 1024 * 1024),
    )(x, g)


# device time: 89971 ns/iter; 3.8199x vs baseline; 1.0008x over previous
import jax
import jax.numpy as jnp
from jax import lax
from jax.experimental import pallas as pl
from jax.experimental.pallas import tpu as pltpu

N_Y = 4
N_P = 8
N_CW = 4
N_CCW = 3
M_BLK = 2048
D = 2048
W = D // N_P
Mh = M_BLK // 2
RS_SUBS = 8
Sh = Mh // RS_SUBS
AG_SUBS = 2
AH = Mh // AG_SUBS
T = 512


def kernel(partial, gamma):
    x = partial.reshape(N_Y * M_BLK, D)
    g = gamma.reshape(1, D)

    def body(x_ref, g_ref, out_ref, rs_send, rs_in, tile, y_all,
             out_stage, load_sems, store_sems, rs_send_sems, rs_recv_sems,
             ag_send_sems, ag_recv_sems):
        my_x = lax.axis_index("x")
        my_y = lax.axis_index("y")
        my_z = lax.axis_index("z")
        y_left = lax.rem(my_y + (N_Y - 1), N_Y)
        y_right = lax.rem(my_y + 1, N_Y)

        p = jnp.where(my_x == 0, my_z, N_P - 1 - my_z)

        def pos_xz(o):
            return jnp.where(o < 4, 0, 1), jnp.where(o < 4, o, N_P - 1 - o)

        def col_of_pos(o):
            ox, oz = pos_xz(o)
            return (ox * 4 + oz) * W

        p_next = lax.rem(p + 1, N_P)
        p_prev = lax.rem(p + (N_P - 1), N_P)
        nx, nz = pos_xz(p_next)
        px, pz = pos_xz(p_prev)
        col0 = (my_x * 4 + my_z) * W

        def rs_chunk(h):
            return lax.rem(my_y + (N_Y - 1 - h), N_Y)

        def load_start(grp, slot, c):
            pltpu.make_async_copy(
                x_ref.at[pl.ds(c * M_BLK + grp * Mh, Mh), pl.ds(col0, W)],
                tile.at[grp, slot], load_sems.at[grp, slot]).start()

        def load_wait(grp, slot):
            pltpu.make_async_copy(
                x_ref.at[pl.ds(0, Mh), pl.ds(col0, W)],
                tile.at[grp, slot], load_sems.at[grp, slot]).wait()

        load_start(0, 0, rs_chunk(0))
        load_start(1, 0, rs_chunk(0))

        barrier_sem = pltpu.get_barrier_semaphore()
        for dev in ((my_x, y_left, my_z), (my_x, y_right, my_z),
                    (nx, my_y, nz), (px, my_y, pz)):
            pl.semaphore_signal(
                barrier_sem, inc=1,
                device_id=dev, device_id_type=pl.DeviceIdType.MESH,
            )
        pl.semaphore_wait(barrier_sem, 4)

        def rs_sub(grp, h, half):
            rsl = pl.ds(half * Sh, Sh)
            return pltpu.make_async_remote_copy(
                src_ref=rs_send.at[grp, rsl],
                dst_ref=rs_in.at[grp, h, rsl],
                send_sem=rs_send_sems.at[grp, h, half],
                recv_sem=rs_recv_sems.at[grp, h, half],
                device_id=(my_x, y_right, my_z),
                device_id_type=pl.DeviceIdType.MESH,
            )

        def rs_hop(grp, h, prev):
            load_wait(grp, h % 2)
            subs = []
            for half in range(RS_SUBS):
                rsl = pl.ds(half * Sh, Sh)
                if h == 0:
                    rs_send[grp, rsl] = tile[grp, 0, rsl].astype(jnp.bfloat16)
                else:
                    prev[half].wait_recv()
                    prev[half].wait_send()
                    rs_send[grp, rsl] = (
                        rs_in[grp, h - 1, rsl].astype(jnp.float32)
                        + tile[grp, h % 2, rsl]
                    ).astype(jnp.bfloat16)
                sub = rs_sub(grp, h, half)
                sub.start()
                subs.append(sub)
            load_start(grp, (h + 1) % 2,
                       rs_chunk(h + 1) if h < N_Y - 2 else my_y)
            return subs

        def rs_finish(grp, prev):
            load_wait(grp, (N_Y - 1) % 2)
            for half in range(RS_SUBS):
                rsl = pl.ds(half * Sh, Sh)
                prev[half].wait_recv()
                prev[half].wait_send()
                y_mine = (rs_in[grp, N_Y - 2, rsl].astype(jnp.float32)
                          + tile[grp, (N_Y - 1) % 2, rsl])
                y_all[pl.ds(grp * Mh + half * Sh, Sh), pl.ds(col0, W)] = (
                    y_mine.astype(jnp.bfloat16))

        def ag_desc(grp, k, dir_idx, sub):
            if dir_idx == 0:
                o = lax.rem(p + (N_P - k), N_P)
                tgt = (nx, my_y, nz)
            else:
                o = lax.rem(p + k, N_P)
                tgt = (px, my_y, pz)
            rsl = pl.ds(grp * Mh + sub * AH, AH)
            csl = pl.ds(col_of_pos(o), W)
            return pltpu.make_async_remote_copy(
                src_ref=y_all.at[rsl, csl],
                dst_ref=y_all.at[rsl, csl],
                send_sem=ag_send_sems.at[grp, k, dir_idx, sub],
                recv_sem=ag_recv_sems.at[grp, k, dir_idx, sub],
                device_id=tgt,
                device_id_type=pl.DeviceIdType.MESH,
            )

        def ag_stage(grp, k, prev):
            pcw, pccw = prev if prev is not None else (None, None)
            cw = [ag_desc(grp, k, 0, s) for s in range(AG_SUBS)]
            ccw = ([ag_desc(grp, k, 1, s) for s in range(AG_SUBS)]
                   if k < N_CCW else None)
            for s in range(AG_SUBS):
                if pcw is not None:
                    pcw[s].wait_recv()
                    pcw[s].wait_send()
                cw[s].start()
                if pccw is not None:
                    pccw[s].wait_recv()
                    pccw[s].wait_send()
                if ccw is not None:
                    ccw[s].start()
            return cw, ccw

        def ag_last_wait(prev):
            pcw, _ = prev
            for s in range(AG_SUBS):
                pcw[s].wait_recv()
                pcw[s].wait_send()

        pending = [False, False]

        def norm_tile(t):
            slot = t % 2
            if pending[slot]:
                pltpu.make_async_copy(
                    out_stage.at[slot], out_ref.at[pl.ds(0, T), :],
                    store_sems.at[slot]).wait()
            rsl = pl.ds(t * T, T)
            y = y_all[rsl, :].astype(jnp.float32)
            ms = jnp.mean(y * y, axis=-1, keepdims=True)
            out_stage[slot] = y * lax.rsqrt(ms + 1e-6) * g_ref[...]
            pltpu.make_async_copy(
                out_stage.at[slot], out_ref.at[rsl, :],
                store_sems.at[slot]).start()
            pending[slot] = True

        prev0 = None
        for h in range(N_Y - 1):
            prev0 = rs_hop(0, h, prev0)
        rs_finish(0, prev0)

        ag0 = None
        ag1 = None
        prev1 = None
        for k in range(N_CW):
            ag0 = ag_stage(0, k, ag0)
            if k <= N_Y - 2:
                prev1 = rs_hop(1, k, prev1)
            else:
                rs_finish(1, prev1)
                ag1 = ag_stage(1, 0, None)

        for k in range(1, N_CW):
            ag1 = ag_stage(1, k, ag1)
            if k == N_CW - 2:
                ag_last_wait(ag0)
                norm_tile(0)
            elif k == N_CW - 1:
                norm_tile(1)
        ag_last_wait(ag1)
        norm_tile(2)
        norm_tile(3)
        for slot in range(2):
            pltpu.make_async_copy(
                out_stage.at[slot], out_ref.at[pl.ds(0, T), :],
                store_sems.at[slot]).wait()

    return pl.pallas_call(
        body,
        out_shape=jax.ShapeDtypeStruct((M_BLK, D), jnp.float32),
        in_specs=[
            pl.BlockSpec(memory_space=pl.ANY),
            pl.BlockSpec(memory_space=pltpu.MemorySpace.VMEM),
        ],
        out_specs=pl.BlockSpec(memory_space=pl.ANY),
        scratch_shapes=[
            pltpu.VMEM((2, Mh, W), jnp.bfloat16),
            pltpu.VMEM((2, N_Y - 1, Mh, W), jnp.bfloat16),
            pltpu.VMEM((2, 2, Mh, W), jnp.float32),
            pltpu.VMEM((M_BLK, D), jnp.bfloat16),
            pltpu.VMEM((2, T, D), jnp.float32),
            pltpu.SemaphoreType.DMA((2, 2)),
            pltpu.SemaphoreType.DMA((2,)),
            pltpu.SemaphoreType.DMA((2, N_Y - 1, RS_SUBS)),
            pltpu.SemaphoreType.DMA((2, N_Y - 1, RS_SUBS)),
            pltpu.SemaphoreType.DMA((2, N_CW, 2, AG_SUBS)),
            pltpu.SemaphoreType.DMA((2, N_CW, 2, AG_SUBS)),
        ],
        compiler_params=pltpu.CompilerParams(
            collective_id=0, vmem_limit_bytes=60 * 1024 * 1024),
    )(x, g)
